# Initial kernel scaffold; baseline (speedup 1.0000x reference)
import jax
import jax.numpy as jnp
from jax.experimental import pallas as pl


def kernel(x, edge_index_sc, edge_weight_sc, edge_index_fc, edge_weight_fc, W_sc0, b_sc0, W_fc0, b_fc0, W_sc1, b_sc1, W_fc1, b_fc1, gamma0, beta0, gamma1, beta1):
    raise NotImplementedError("write your pallas kernel here")


# speedup vs baseline: 11.0355x; 11.0355x over previous
import functools

import jax
import jax.numpy as jnp
from jax import lax
from jax.experimental import pallas as pl
from jax.experimental.pallas import tpu as pltpu
from jax.experimental.pallas import tpu_sc as plsc

N = 10000
E = 320000
D = 128
H = 128

NC = 2
NS = 16
NW = NC * NS
L = 16
C = 128
NCHUNK = E // C
NPAD = 10240
RPT = NPAD // NS
ZR = 160

_mesh = plsc.VectorSubcoreMesh(
    core_axis_name="c", subcore_axis_name="s", num_cores=NC, num_subcores=NS
)


def _rsqrt_nr(v):
    y = 1.0 / v
    for _ in range(28):
        y = y * (1.5 - 0.5 * v * y * y)
    return y


@functools.partial(
    pl.kernel,
    out_type=[
        jax.ShapeDtypeStruct((E,), jnp.float32),
        jax.ShapeDtypeStruct((E,), jnp.float32),
        jax.ShapeDtypeStruct((2, NPAD), jnp.float32),
    ],
    mesh=_mesh,
    compiler_params=pltpu.CompilerParams(needs_layout_passes=False),
    scratch_types=[
        pltpu.VMEM_SHARED((NPAD,), jnp.float32),
        pltpu.VMEM_SHARED((NPAD,), jnp.float32),
        pltpu.VMEM_SHARED((NPAD,), jnp.float32),
        pltpu.VMEM_SHARED((NPAD,), jnp.float32),
        pltpu.VMEM((NPAD,), jnp.float32),
        pltpu.VMEM((NPAD,), jnp.float32),
        pltpu.VMEM((C,), jnp.int32),
        pltpu.VMEM((C,), jnp.int32),
        pltpu.VMEM((C,), jnp.float32),
        pltpu.VMEM((C,), jnp.float32),
        pltpu.VMEM((RPT,), jnp.float32),
        pltpu.VMEM((RPT,), jnp.float32),
    ],
)
def _norm_kernel(ei_sc, w_sc, ei_fc, w_fc, norm_sc, norm_fc, dsq,
                 deg_a, deg_b, dinv_a, dinv_b, dloc_a, dloc_b,
                 row_b, col_b, w_b, nrm_b, dbuf, sqbuf):
    c = lax.axis_index("c")
    s = lax.axis_index("s")
    wid = s * NC + c
    off = s * RPT

    def zfill(i, _):
        dbuf[pl.ds(i * L, L)] = jnp.zeros((L,), jnp.float32)
        return 0
    lax.fori_loop(0, RPT // L, zfill, 0)
    pltpu.sync_copy(dbuf, deg_a.at[pl.ds(off, RPT)])
    pltpu.sync_copy(dbuf, deg_b.at[pl.ds(off, RPT)])
    plsc.subcore_barrier()

    def deg_pass(ei, w, deg):
        def body(t, _):
            j = s + NS * t

            @pl.when(j < NCHUNK)
            def _():
                pltpu.sync_copy(ei.at[1, pl.ds(j * C, C)], col_b)
                pltpu.sync_copy(w.at[pl.ds(j * C, C)], w_b)
                pltpu.sync_copy(w_b, deg.at[col_b], add=True)
            return 0
        lax.fori_loop(0, (NCHUNK + NS - 1) // NS, body, 0)

    deg_pass(ei_sc, w_sc, deg_a)
    deg_pass(ei_fc, w_fc, deg_b)
    plsc.subcore_barrier()

    def dinv_pass(deg, dinv, gi):
        pltpu.sync_copy(deg.at[pl.ds(off, RPT)], dbuf)

        def body(i, _):
            v = dbuf[pl.ds(i * L, L)] + 1.0
            y = _rsqrt_nr(v)
            dbuf[pl.ds(i * L, L)] = y
            sqbuf[pl.ds(i * L, L)] = y * y
            return 0
        lax.fori_loop(0, RPT // L, body, 0)
        pltpu.sync_copy(dbuf, dinv.at[pl.ds(off, RPT)])

        @pl.when(c == 0)
        def _():
            pltpu.sync_copy(sqbuf, dsq.at[gi, pl.ds(off, RPT)])

    dinv_pass(deg_a, dinv_a, 0)
    dinv_pass(deg_b, dinv_b, 1)
    plsc.subcore_barrier()

    pltpu.sync_copy(dinv_a, dloc_a)
    pltpu.sync_copy(dinv_b, dloc_b)

    def norm_pass(ei, w, dloc, norm_out):
        def body(t, _):
            j = wid + NW * t

            @pl.when(j < NCHUNK)
            def _():
                pltpu.sync_copy(ei.at[0, pl.ds(j * C, C)], row_b)
                pltpu.sync_copy(ei.at[1, pl.ds(j * C, C)], col_b)
                pltpu.sync_copy(w.at[pl.ds(j * C, C)], w_b)
                for k in range(C // L):
                    ir = row_b[pl.ds(k * L, L)]
                    ic = col_b[pl.ds(k * L, L)]
                    dr = plsc.load_gather(dloc, [ir])
                    dc = plsc.load_gather(dloc, [ic])
                    nrm_b[pl.ds(k * L, L)] = dr * w_b[pl.ds(k * L, L)] * dc
                pltpu.sync_copy(nrm_b, norm_out.at[pl.ds(j * C, C)])
            return 0
        lax.fori_loop(0, (NCHUNK + NW - 1) // NW, body, 0)

    norm_pass(ei_sc, w_sc, dloc_a, norm_sc)
    norm_pass(ei_fc, w_fc, dloc_b, norm_fc)


@functools.partial(
    pl.kernel,
    out_type=jax.ShapeDtypeStruct((NC, NPAD, H), jnp.float32),
    mesh=_mesh,
    compiler_params=pltpu.CompilerParams(needs_layout_passes=False),
    scratch_types=[
        pltpu.VMEM_SHARED((NPAD, H), jnp.float32),
        pltpu.VMEM((2, C), jnp.int32),
        pltpu.VMEM((2, C), jnp.int32),
        pltpu.VMEM((2, C), jnp.float32),
        pltpu.VMEM((2, C, H), jnp.float32),
        pltpu.SemaphoreType.DMA,
        pltpu.SemaphoreType.DMA,
    ],
)
def _spmm_kernel(ei, nrm, hmat, part,
                 acc, row_b, col_b, nrm_b, rows, sem0, sem1):
    c = lax.axis_index("c")
    s = lax.axis_index("s")
    wid = s * NC + c
    sems = (sem0, sem1)

    def zfill(i, _):
        for k in range(H // L):
            rows[0, i, pl.ds(k * L, L)] = jnp.zeros((L,), jnp.float32)
        return 0
    lax.fori_loop(0, C, zfill, 0)
    for r in range(RPT // C):
        pltpu.sync_copy(rows.at[0], acc.at[pl.ds(s * RPT + r * C, C), :])
    plsc.subcore_barrier()

    def stage(j, b):
        pltpu.sync_copy(ei.at[0, pl.ds(j * C, C)], row_b.at[b])
        pltpu.sync_copy(ei.at[1, pl.ds(j * C, C)], col_b.at[b])
        pltpu.sync_copy(nrm.at[pl.ds(j * C, C)], nrm_b.at[b])
        return pltpu.async_copy(hmat.at[row_b.at[b]], rows.at[b], sems[b])

    def consume(b, cp):
        cp.wait()

        def scale(g, _):
            nv = nrm_b[b, pl.ds(g * L, L)]
            base = g * L
            for ii in range(L):
                sc = nv[ii]
                for k in range(H // L):
                    rows[b, base + ii, pl.ds(k * L, L)] = (
                        rows[b, base + ii, pl.ds(k * L, L)] * sc)
            return 0
        lax.fori_loop(0, C // L, scale, 0)
        pltpu.sync_copy(rows.at[b], acc.at[col_b.at[b]], add=True)

    def outer(t2, _):
        j0 = wid + NW * (2 * t2)
        j1 = j0 + NW
        cp0 = stage(j0, 0)
        cp1 = stage(j1, 1)
        consume(0, cp0)
        consume(1, cp1)
        return 0
    lax.fori_loop(0, 39, outer, 0)

    @pl.when(wid < NCHUNK - 78 * NW)
    def _():
        cp = stage(78 * NW + wid, 0)
        consume(0, cp)

    plsc.subcore_barrier()
    pltpu.sync_copy(acc.at[pl.ds(s * RPT, RPT), :],
                    part.at[c, pl.ds(s * RPT, RPT), :])


def _mm2_body(x_ref, wa_ref, wb_ref, oa_ref, ob_ref):
    xv = x_ref[...]
    oa_ref[...] = jnp.dot(xv, wa_ref[...], preferred_element_type=jnp.float32)
    ob_ref[...] = jnp.dot(xv, wb_ref[...], preferred_element_type=jnp.float32)


_mm2 = pl.pallas_call(
    _mm2_body,
    out_shape=[jax.ShapeDtypeStruct((N, H), jnp.float32)] * 2,
)


def _fuse(psc, pfc, hs, hf, dss, dsf, bs, bf, g, be):
    y = (psc[0, :N, :] + psc[1, :N, :] + dss[...] * hs[...] + bs[...]
         + pfc[0, :N, :] + pfc[1, :N, :] + dsf[...] * hf[...] + bf[...])
    m = jnp.mean(y, axis=0, keepdims=True)
    v = jnp.mean((y - m) * (y - m), axis=0, keepdims=True)
    return jnp.maximum((y - m) * lax.rsqrt(v + 1e-5) * g[...] + be[...], 0.0)


def _combine_mm_body(psc, pfc, hs, hf, dss, dsf, bs, bf, g, be,
                     ws1, wf1, os1, of1):
    h1 = _fuse(psc, pfc, hs, hf, dss, dsf, bs, bf, g, be)
    os1[...] = jnp.dot(h1, ws1[...], preferred_element_type=jnp.float32)
    of1[...] = jnp.dot(h1, wf1[...], preferred_element_type=jnp.float32)


_combine_mm = pl.pallas_call(
    _combine_mm_body,
    out_shape=[jax.ShapeDtypeStruct((N, H), jnp.float32)] * 2,
)


def _combine_body(psc, pfc, hs, hf, dss, dsf, bs, bf, g, be, o):
    o[...] = _fuse(psc, pfc, hs, hf, dss, dsf, bs, bf, g, be)


_combine = pl.pallas_call(
    _combine_body,
    out_shape=jax.ShapeDtypeStruct((N, H), jnp.float32),
)


def kernel(x, edge_index_sc, edge_weight_sc, edge_index_fc, edge_weight_fc,
           W_sc0, b_sc0, W_fc0, b_fc0, W_sc1, b_sc1, W_fc1, b_fc1,
           gamma0, beta0, gamma1, beta1):
    norm_sc, norm_fc, dsq = _norm_kernel(
        edge_index_sc, edge_weight_sc, edge_index_fc, edge_weight_fc)
    dss = dsq[0, :N].reshape(N, 1)
    dsf = dsq[1, :N].reshape(N, 1)

    r = lambda a: a.reshape(1, H)
    hs0, hf0 = _mm2(x, W_sc0, W_fc0)
    ps0 = _spmm_kernel(edge_index_sc, norm_sc, hs0)
    pf0 = _spmm_kernel(edge_index_fc, norm_fc, hf0)
    hs1, hf1 = _combine_mm(ps0, pf0, hs0, hf0, dss, dsf,
                           r(b_sc0), r(b_fc0), r(gamma0), r(beta0),
                           W_sc1, W_fc1)
    ps1 = _spmm_kernel(edge_index_sc, norm_sc, hs1)
    pf1 = _spmm_kernel(edge_index_fc, norm_fc, hf1)
    return _combine(ps1, pf1, hs1, hf1, dss, dsf,
                    r(b_sc1), r(b_fc1), r(gamma1), r(beta1))
